# Initial kernel scaffold; baseline (speedup 1.0000x reference)
import jax
import jax.numpy as jnp
from jax.experimental import pallas as pl


def kernel(masks_a, masks_b, scores_a, scores_b):
    raise NotImplementedError("write your pallas kernel here")


# speedup vs baseline: 1.3977x; 1.3977x over previous
import functools

import jax
import jax.numpy as jnp
from jax.experimental import pallas as pl
from jax.experimental.pallas import tpu as pltpu

N = 16
K = 512 * 512
BK = 8192
THR = 0.8


def _reduce_kernel(a_ref, b_ref, inter_ref, aa_ref, ab_ref):
    k = pl.program_id(0)
    a = a_ref[...]
    b = b_ref[...]
    part = jax.lax.dot_general(
        a, b, (((1,), (1,)), ((), ())), preferred_element_type=jnp.float32
    )
    pa = jnp.sum(a, axis=1, keepdims=True)
    pb = jnp.sum(b, axis=1, keepdims=True)

    @pl.when(k == 0)
    def _():
        inter_ref[...] = part
        aa_ref[...] = pa
        ab_ref[...] = pb

    @pl.when(k != 0)
    def _():
        inter_ref[...] += part
        aa_ref[...] += pa
        ab_ref[...] += pb


def _apply_kernel(inter_ref, aa_ref, ab_ref, sa_ref, sb_ref, a_ref, b_ref, out_ref):
    inter = inter_ref[...]
    area_a = aa_ref[...]
    area_b = ab_ref[...]
    union = area_a + area_b.T - inter
    iou = jnp.where(union > 0, inter / jnp.maximum(union, 1e-8), 0.0)

    a_loses = sa_ref[...] <= sb_ref[...].T
    high = iou > THR
    partial = (iou > 0) & (~high)

    keep_a = 1.0 - jnp.any(high & a_loses, axis=1, keepdims=True).astype(jnp.float32)
    keep_b = 1.0 - jnp.any(high & (~a_loses), axis=0, keepdims=True).astype(
        jnp.float32
    ).T
    w_a = (partial & a_loses).astype(jnp.float32)
    w_b = (partial & (~a_loses)).astype(jnp.float32)

    a = a_ref[...]
    b = b_ref[...]
    region_a = jax.lax.dot_general(
        w_a, b, (((1,), (0,)), ((), ())), preferred_element_type=jnp.float32
    )
    region_b = jax.lax.dot_general(
        w_b, a, (((0,), (0,)), ((), ())), preferred_element_type=jnp.float32
    )
    out_ref[0] = jnp.where(region_a > 0, 0.0, a) * keep_a
    out_ref[1] = jnp.where(region_b > 0, 0.0, b) * keep_b


def _run(masks_a, masks_b, scores_a, scores_b):
    a2 = masks_a.reshape(N, K)
    b2 = masks_b.reshape(N, K)
    sa = scores_a.reshape(N, 1)
    sb = scores_b.reshape(N, 1)
    nblk = K // BK

    inter, aa, ab = pl.pallas_call(
        _reduce_kernel,
        grid=(nblk,),
        in_specs=[
            pl.BlockSpec((N, BK), lambda k: (0, k)),
            pl.BlockSpec((N, BK), lambda k: (0, k)),
        ],
        out_specs=[
            pl.BlockSpec((N, N), lambda k: (0, 0)),
            pl.BlockSpec((N, 1), lambda k: (0, 0)),
            pl.BlockSpec((N, 1), lambda k: (0, 0)),
        ],
        out_shape=[
            jax.ShapeDtypeStruct((N, N), jnp.float32),
            jax.ShapeDtypeStruct((N, 1), jnp.float32),
            jax.ShapeDtypeStruct((N, 1), jnp.float32),
        ],
    )(a2, b2)

    out = pl.pallas_call(
        _apply_kernel,
        grid=(nblk,),
        in_specs=[
            pl.BlockSpec((N, N), lambda k: (0, 0)),
            pl.BlockSpec((N, 1), lambda k: (0, 0)),
            pl.BlockSpec((N, 1), lambda k: (0, 0)),
            pl.BlockSpec((N, 1), lambda k: (0, 0)),
            pl.BlockSpec((N, 1), lambda k: (0, 0)),
            pl.BlockSpec((N, BK), lambda k: (0, k)),
            pl.BlockSpec((N, BK), lambda k: (0, k)),
        ],
        out_specs=pl.BlockSpec((2, N, BK), lambda k: (0, 0, k)),
        out_shape=jax.ShapeDtypeStruct((2, N, K), jnp.float32),
    )(inter, aa, ab, sa, sb, a2, b2)

    return out.reshape(2, N, 512, 512)


@jax.jit
def kernel(masks_a, masks_b, scores_a, scores_b):
    return _run(masks_a, masks_b, scores_a, scores_b)
